# baseline (speedup 1.0000x reference)
import jax
import jax.numpy as jnp
from jax import lax
from jax.experimental import pallas as pl
from jax.experimental.pallas import tpu as pltpu
from jax.experimental.pallas import tpu_sc as plsc

BOX = 120
D = 2
NMOL = 22
NATOM = 400
SLABW = 4
NSLAB = BOX // SLABW
BOXP = 128
PLANE = (BOX // 8) * 8 * BOXP
SLABWORDS = SLABW * PLANE
NT = NMOL * NSLAB
NC = 2
NS = 16
NW = NC * NS
KMAX = (NT + NW - 1) // NW
ZHALF = SLABWORDS * 3 // 4


def _body(coords_hbm, na_hbm, out_hbm, cv, na_v, sel, buf0, buf1, zsh,
          sem0, sem1, semz):
    c = lax.axis_index("c")
    s = lax.axis_index("s")
    wid = s * NC + c

    pltpu.sync_copy(na_hbm, na_v)

    lane = lax.iota(jnp.int32, 16)
    zero16 = jnp.zeros((16,), jnp.float32)
    oy1 = lane // 5
    oz1 = lane - oy1 * 5
    l2 = lane + 16
    oy2 = l2 // 5
    oz2 = l2 - oy2 * 5
    m_l2 = l2 < 25
    oy1f = oy1.astype(jnp.float32)
    oz1f = oz1.astype(jnp.float32)
    oy2f = oy2.astype(jnp.float32)
    oz2f = oz2.astype(jnp.float32)
    i0 = jnp.zeros((16,), jnp.int32)
    i1 = i0 + 1
    i2 = i0 + 2

    stripe = ZHALF // NS

    def z0(j, carry):
        buf0[pl.ds(j * 16, 16)] = zero16
        return carry

    lax.fori_loop(0, stripe // 16, z0, 0)
    pltpu.sync_copy(buf0.at[pl.ds(0, stripe)], zsh.at[pl.ds(s * stripe, stripe)])
    plsc.subcore_barrier()

    def process(k, t, buf, sem):
        if True:
            m = t // NSLAB
            slab = t - m * NSLAB
            x0 = slab * SLABW

            @pl.when((k == 0) | (slab == 0))
            def _():
                pltpu.sync_copy(coords_hbm.at[m], cv)

            @pl.when(k >= 2)
            def _():
                pltpu.make_async_copy(
                    buf, out_hbm.at[pl.ds(0, SLABWORDS)], sem).wait()
            hz = pltpu.async_copy(zsh, buf.at[pl.ds(0, ZHALF)], semz)
            na = plsc.load_gather(na_v, [i0 + m])

            def zb(j, carry):
                for u in range(8):
                    buf[pl.ds(ZHALF + (j * 8 + u) * 16, 16)] = zero16
                return carry

            lax.fori_loop(0, (SLABWORDS - ZHALF) // 128, zb, 0)

            def selb(i, cnt):
                ids = i * 16 + lane
                xs = plsc.load_gather(cv, [ids * 3])
                cx = xs.astype(jnp.int32)
                msk = (ids < na) & (cx >= x0 - D) & (cx <= x0 + SLABW - 1 + D)
                plsc.store_compressed(sel.at[pl.ds(cnt, 16)], ids, mask=msk)
                pc = plsc.all_reduce_population_count(msk)
                return cnt + jnp.max(pc)

            cnt = lax.fori_loop(0, NATOM // 16, selb, 0)
            hz.wait()

            def atom(i, carry):
                aid3 = plsc.load_gather(sel, [i0 + i]) * 3
                x = plsc.load_gather(cv, [aid3])
                y = plsc.load_gather(cv, [aid3 + 1])
                z = plsc.load_gather(cv, [aid3 + 2])
                cx = jnp.clip(x.astype(jnp.int32), D, BOX - 1 - D)
                cy = jnp.clip(y.astype(jnp.int32), D, BOX - 1 - D)
                cz = jnp.clip(z.astype(jnp.int32), D, BOX - 1 - D)
                cyf = cy.astype(jnp.float32)
                czf = cz.astype(jnp.float32)
                dy0 = y - cyf + jnp.float32(D)
                dz0 = z - czf + jnp.float32(D)
                dy1 = dy0 - oy1f
                dz1 = dz0 - oz1f
                wyz1 = jnp.exp(-(dy1 * dy1 + dz1 * dz1))
                dy2 = dy0 - oy2f
                dz2 = dz0 - oz2f
                wyz2 = jnp.exp(-(dy2 * dy2 + dz2 * dz2))
                ny1 = cy - D + oy1
                nz1 = cz - D + oz1
                ny2 = cy - D + oy2
                nz2 = cz - D + oz2
                base1 = (ny1 >> 3) * 1024 + (ny1 & 7) * BOXP + nz1
                base2 = (ny2 >> 3) * 1024 + (ny2 & 7) * BOXP + nz2
                dx0 = x - cx.astype(jnp.float32) + jnp.float32(D)
                xrel0 = cx - D - x0
                for ox in range(2 * D + 1):
                    dx = dx0 - jnp.float32(ox)
                    wx = jnp.exp(-(dx * dx))
                    xr = xrel0 + ox
                    ms = (xr >= 0) & (xr < SLABW)
                    xrc = jnp.clip(xr, 0, SLABW - 1)
                    row = xrc * PLANE
                    plsc.addupdate_scatter(buf, [row + base1], wx * wyz1,
                                           mask=ms)
                    plsc.addupdate_scatter(buf, [row + base2], wx * wyz2,
                                           mask=ms & m_l2)
                return carry

            lax.fori_loop(0, cnt, atom, 0)

            pltpu.async_copy(
                buf, out_hbm.at[pl.ds(t * SLABWORDS, SLABWORDS)], sem)

    start = 20 * wid + jnp.minimum(wid, 20)
    ntask = jnp.where(wid < 20, 21, 20)

    def kstep(k, carry):
        t = start + k
        par = lax.rem(k, 2)

        @pl.when((k < ntask) & (par == 0))
        def _():
            process(k, t, buf0, sem0)

        @pl.when((k < ntask) & (par == 1))
        def _():
            process(k, t, buf1, sem1)

        return carry

    lax.fori_loop(0, KMAX, kstep, 0)
    pltpu.make_async_copy(buf0, out_hbm.at[pl.ds(0, SLABWORDS)], sem0).wait()
    pltpu.make_async_copy(buf1, out_hbm.at[pl.ds(0, SLABWORDS)], sem1).wait()


def kernel(input_coords, num_atoms):
    b, t, mc = input_coords.shape
    n_max = mc // 3
    coords = input_coords.reshape(b * t, n_max * 3)
    na = num_atoms.reshape(b * t).astype(jnp.int32)
    na = jnp.pad(na, (0, 32 - b * t))

    mesh = plsc.VectorSubcoreMesh(core_axis_name="c", subcore_axis_name="s")
    run = pl.kernel(
        _body,
        out_type=jax.ShapeDtypeStruct((NT * SLABWORDS,), jnp.float32),
        mesh=mesh,
        compiler_params=pltpu.CompilerParams(needs_layout_passes=False),
        scratch_types=[
            pltpu.VMEM((3 * NATOM,), jnp.float32),
            pltpu.VMEM((32,), jnp.int32),
            pltpu.VMEM((NATOM + 32,), jnp.int32),
            pltpu.VMEM((SLABWORDS,), jnp.float32),
            pltpu.VMEM((SLABWORDS,), jnp.float32),
            pltpu.VMEM_SHARED((ZHALF,), jnp.float32),
            pltpu.SemaphoreType.DMA,
            pltpu.SemaphoreType.DMA,
            pltpu.SemaphoreType.DMA,
        ],
    )
    vols = run(coords, na)
    vols = vols.reshape(b, t, BOX, BOX, BOXP)[..., :BOX]
    return vols


# speedup vs baseline: 1.0848x; 1.0848x over previous
import jax
import jax.numpy as jnp
from jax import lax
from jax.experimental import pallas as pl
from jax.experimental.pallas import tpu as pltpu
from jax.experimental.pallas import tpu_sc as plsc

BOX = 120
D = 2
NMOL = 22
NATOM = 400
SLABW = 4
NSLAB = BOX // SLABW
BOXP = 128
PLANE = (BOX // 8) * 8 * BOXP
SLABWORDS = SLABW * PLANE
NT = NMOL * NSLAB
NC = 2
NS = 16
NW = NC * NS
KMAX = (NT + NW - 1) // NW
ZHALF = SLABWORDS * 3 // 5


def _body(coords_hbm, na_hbm, out_hbm, cv, na_v, sel, buf0, buf1, zsh,
          sem0, sem1, semz):
    c = lax.axis_index("c")
    s = lax.axis_index("s")
    wid = s * NC + c

    pltpu.sync_copy(na_hbm, na_v)

    lane = lax.iota(jnp.int32, 16)
    zero16 = jnp.zeros((16,), jnp.float32)
    oy1 = lane // 5
    oz1 = lane - oy1 * 5
    l2 = lane + 16
    oy2 = l2 // 5
    oz2 = l2 - oy2 * 5
    m_l2 = l2 < 25
    oy1f = oy1.astype(jnp.float32)
    oz1f = oz1.astype(jnp.float32)
    oy2f = oy2.astype(jnp.float32)
    oz2f = oz2.astype(jnp.float32)
    i0 = jnp.zeros((16,), jnp.int32)
    i1 = i0 + 1
    i2 = i0 + 2

    stripe = ZHALF // NS

    def z0(j, carry):
        buf0[pl.ds(j * 16, 16)] = zero16
        return carry

    lax.fori_loop(0, stripe // 16, z0, 0)
    pltpu.sync_copy(buf0.at[pl.ds(0, stripe)], zsh.at[pl.ds(s * stripe, stripe)])
    plsc.subcore_barrier()

    def process(k, t, buf, sem):
        if True:
            m = t // NSLAB
            slab = t - m * NSLAB
            x0 = slab * SLABW

            @pl.when((k == 0) | (slab == 0))
            def _():
                pltpu.sync_copy(coords_hbm.at[m], cv)

            @pl.when(k >= 2)
            def _():
                pltpu.make_async_copy(
                    buf, out_hbm.at[pl.ds(0, SLABWORDS)], sem).wait()
            hz = pltpu.async_copy(zsh, buf.at[pl.ds(0, ZHALF)], semz)
            na = plsc.load_gather(na_v, [i0 + m])

            def zb(j, carry):
                for u in range(8):
                    buf[pl.ds(ZHALF + (j * 8 + u) * 16, 16)] = zero16
                return carry

            lax.fori_loop(0, (SLABWORDS - ZHALF) // 128, zb, 0)

            def selb(i, cnt):
                ids = i * 16 + lane
                xs = plsc.load_gather(cv, [ids * 3])
                cx = xs.astype(jnp.int32)
                msk = (ids < na) & (cx >= x0 - D) & (cx <= x0 + SLABW - 1 + D)
                plsc.store_compressed(sel.at[pl.ds(cnt, 16)], ids, mask=msk)
                pc = plsc.all_reduce_population_count(msk)
                return cnt + jnp.max(pc)

            cnt = lax.fori_loop(0, NATOM // 16, selb, 0)
            hz.wait()

            def atom(i, carry):
                aid3 = plsc.load_gather(sel, [i0 + i]) * 3
                x = plsc.load_gather(cv, [aid3])
                y = plsc.load_gather(cv, [aid3 + 1])
                z = plsc.load_gather(cv, [aid3 + 2])
                cx = jnp.clip(x.astype(jnp.int32), D, BOX - 1 - D)
                cy = jnp.clip(y.astype(jnp.int32), D, BOX - 1 - D)
                cz = jnp.clip(z.astype(jnp.int32), D, BOX - 1 - D)
                cyf = cy.astype(jnp.float32)
                czf = cz.astype(jnp.float32)
                dy0 = y - cyf + jnp.float32(D)
                dz0 = z - czf + jnp.float32(D)
                dy1 = dy0 - oy1f
                dz1 = dz0 - oz1f
                wyz1 = jnp.exp(-(dy1 * dy1 + dz1 * dz1))
                dy2 = dy0 - oy2f
                dz2 = dz0 - oz2f
                wyz2 = jnp.exp(-(dy2 * dy2 + dz2 * dz2))
                ny1 = cy - D + oy1
                nz1 = cz - D + oz1
                ny2 = cy - D + oy2
                nz2 = cz - D + oz2
                base1 = (ny1 >> 3) * 1024 + (ny1 & 7) * BOXP + nz1
                base2 = (ny2 >> 3) * 1024 + (ny2 & 7) * BOXP + nz2
                dx0 = x - cx.astype(jnp.float32) + jnp.float32(D)
                xrel0 = cx - D - x0
                for ox in range(2 * D + 1):
                    dx = dx0 - jnp.float32(ox)
                    wx = jnp.exp(-(dx * dx))
                    xr = xrel0 + ox
                    ms = (xr >= 0) & (xr < SLABW)
                    xrc = jnp.clip(xr, 0, SLABW - 1)
                    row = xrc * PLANE
                    plsc.addupdate_scatter(buf, [row + base1], wx * wyz1,
                                           mask=ms)
                    plsc.addupdate_scatter(buf, [row + base2], wx * wyz2,
                                           mask=ms & m_l2)
                return carry

            lax.fori_loop(0, cnt, atom, 0)

            pltpu.async_copy(
                buf, out_hbm.at[pl.ds(t * SLABWORDS, SLABWORDS)], sem)

    start = 20 * wid + jnp.minimum(wid, 20)
    ntask = jnp.where(wid < 20, 21, 20)

    def kstep(k, carry):
        t = start + k
        par = lax.rem(k, 2)

        @pl.when((k < ntask) & (par == 0))
        def _():
            process(k, t, buf0, sem0)

        @pl.when((k < ntask) & (par == 1))
        def _():
            process(k, t, buf1, sem1)

        return carry

    lax.fori_loop(0, KMAX, kstep, 0)
    pltpu.make_async_copy(buf0, out_hbm.at[pl.ds(0, SLABWORDS)], sem0).wait()
    pltpu.make_async_copy(buf1, out_hbm.at[pl.ds(0, SLABWORDS)], sem1).wait()


def kernel(input_coords, num_atoms):
    b, t, mc = input_coords.shape
    n_max = mc // 3
    coords = input_coords.reshape(b * t, n_max * 3)
    na = num_atoms.reshape(b * t).astype(jnp.int32)
    na = jnp.pad(na, (0, 32 - b * t))

    mesh = plsc.VectorSubcoreMesh(core_axis_name="c", subcore_axis_name="s")
    run = pl.kernel(
        _body,
        out_type=jax.ShapeDtypeStruct((NT * SLABWORDS,), jnp.float32),
        mesh=mesh,
        compiler_params=pltpu.CompilerParams(needs_layout_passes=False),
        scratch_types=[
            pltpu.VMEM((3 * NATOM,), jnp.float32),
            pltpu.VMEM((32,), jnp.int32),
            pltpu.VMEM((NATOM + 32,), jnp.int32),
            pltpu.VMEM((SLABWORDS,), jnp.float32),
            pltpu.VMEM((SLABWORDS,), jnp.float32),
            pltpu.VMEM_SHARED((ZHALF,), jnp.float32),
            pltpu.SemaphoreType.DMA,
            pltpu.SemaphoreType.DMA,
            pltpu.SemaphoreType.DMA,
        ],
    )
    vols = run(coords, na)
    vols = vols.reshape(b, t, BOX, BOX, BOXP)[..., :BOX]
    return vols


# speedup vs baseline: 1.0966x; 1.0109x over previous
import jax
import jax.numpy as jnp
from jax import lax
from jax.experimental import pallas as pl
from jax.experimental.pallas import tpu as pltpu
from jax.experimental.pallas import tpu_sc as plsc

BOX = 120
D = 2
NMOL = 22
NATOM = 400
SLABW = 4
NSLAB = BOX // SLABW
BOXP = 128
PLANE = (BOX // 8) * 8 * BOXP
SLABWORDS = SLABW * PLANE
NT = NMOL * NSLAB
NC = 2
NS = 16
NW = NC * NS
KMAX = (NT + NW - 1) // NW
ZHALF = SLABWORDS // 2


def _body(coords_hbm, na_hbm, out_hbm, cv, na_v, sel, buf0, buf1, zsh,
          sem0, sem1, semz):
    c = lax.axis_index("c")
    s = lax.axis_index("s")
    wid = s * NC + c

    pltpu.sync_copy(na_hbm, na_v)

    lane = lax.iota(jnp.int32, 16)
    zero16 = jnp.zeros((16,), jnp.float32)
    oy1 = lane // 5
    oz1 = lane - oy1 * 5
    l2 = lane + 16
    oy2 = l2 // 5
    oz2 = l2 - oy2 * 5
    m_l2 = l2 < 25
    oy1f = oy1.astype(jnp.float32)
    oz1f = oz1.astype(jnp.float32)
    oy2f = oy2.astype(jnp.float32)
    oz2f = oz2.astype(jnp.float32)
    i0 = jnp.zeros((16,), jnp.int32)
    i1 = i0 + 1
    i2 = i0 + 2

    stripe = ZHALF // NS

    def z0(j, carry):
        buf0[pl.ds(j * 16, 16)] = zero16
        return carry

    lax.fori_loop(0, stripe // 16, z0, 0)
    pltpu.sync_copy(buf0.at[pl.ds(0, stripe)], zsh.at[pl.ds(s * stripe, stripe)])
    plsc.subcore_barrier()

    def process(k, t, buf, sem):
        if True:
            m = t // NSLAB
            slab = t - m * NSLAB
            x0 = slab * SLABW

            @pl.when((k == 0) | (slab == 0))
            def _():
                pltpu.sync_copy(coords_hbm.at[m], cv)

            @pl.when(k >= 2)
            def _():
                pltpu.make_async_copy(
                    buf, out_hbm.at[pl.ds(0, SLABWORDS)], sem).wait()
            hz = pltpu.async_copy(zsh, buf.at[pl.ds(0, ZHALF)], semz)
            na = plsc.load_gather(na_v, [i0 + m])

            def zb(j, carry):
                for u in range(8):
                    buf[pl.ds(ZHALF + (j * 8 + u) * 16, 16)] = zero16
                return carry

            lax.fori_loop(0, (SLABWORDS - ZHALF) // 128, zb, 0)

            def selb(i, cnt):
                ids = i * 16 + lane
                xs = plsc.load_gather(cv, [ids * 3])
                cx = xs.astype(jnp.int32)
                msk = (ids < na) & (cx >= x0 - D) & (cx <= x0 + SLABW - 1 + D)
                plsc.store_compressed(sel.at[pl.ds(cnt, 16)], ids, mask=msk)
                pc = plsc.all_reduce_population_count(msk)
                return cnt + jnp.max(pc)

            cnt = lax.fori_loop(0, NATOM // 16, selb, 0)
            hz.wait()

            def atom(i, carry):
                aid3 = plsc.load_gather(sel, [i0 + i]) * 3
                x = plsc.load_gather(cv, [aid3])
                y = plsc.load_gather(cv, [aid3 + 1])
                z = plsc.load_gather(cv, [aid3 + 2])
                cx = jnp.clip(x.astype(jnp.int32), D, BOX - 1 - D)
                cy = jnp.clip(y.astype(jnp.int32), D, BOX - 1 - D)
                cz = jnp.clip(z.astype(jnp.int32), D, BOX - 1 - D)
                cyf = cy.astype(jnp.float32)
                czf = cz.astype(jnp.float32)
                dy0 = y - cyf + jnp.float32(D)
                dz0 = z - czf + jnp.float32(D)
                dy1 = dy0 - oy1f
                dz1 = dz0 - oz1f
                wyz1 = jnp.exp(-(dy1 * dy1 + dz1 * dz1))
                dy2 = dy0 - oy2f
                dz2 = dz0 - oz2f
                wyz2 = jnp.exp(-(dy2 * dy2 + dz2 * dz2))
                ny1 = cy - D + oy1
                nz1 = cz - D + oz1
                ny2 = cy - D + oy2
                nz2 = cz - D + oz2
                base1 = (ny1 >> 3) * 1024 + (ny1 & 7) * BOXP + nz1
                base2 = (ny2 >> 3) * 1024 + (ny2 & 7) * BOXP + nz2
                dx0 = x - cx.astype(jnp.float32) + jnp.float32(D)
                xrel0 = cx - D - x0
                for ox in range(2 * D + 1):
                    dx = dx0 - jnp.float32(ox)
                    wx = jnp.exp(-(dx * dx))
                    xr = xrel0 + ox
                    ms = (xr >= 0) & (xr < SLABW)
                    xrc = jnp.clip(xr, 0, SLABW - 1)
                    row = xrc * PLANE
                    plsc.addupdate_scatter(buf, [row + base1], wx * wyz1,
                                           mask=ms)
                    plsc.addupdate_scatter(buf, [row + base2], wx * wyz2,
                                           mask=ms & m_l2)
                return carry

            lax.fori_loop(0, cnt, atom, 0)

            pltpu.async_copy(
                buf, out_hbm.at[pl.ds(t * SLABWORDS, SLABWORDS)], sem)

    start = 20 * wid + jnp.minimum(wid, 20)
    ntask = jnp.where(wid < 20, 21, 20)

    def kstep(k, carry):
        t = start + k
        par = lax.rem(k, 2)

        @pl.when((k < ntask) & (par == 0))
        def _():
            process(k, t, buf0, sem0)

        @pl.when((k < ntask) & (par == 1))
        def _():
            process(k, t, buf1, sem1)

        return carry

    lax.fori_loop(0, KMAX, kstep, 0)
    pltpu.make_async_copy(buf0, out_hbm.at[pl.ds(0, SLABWORDS)], sem0).wait()
    pltpu.make_async_copy(buf1, out_hbm.at[pl.ds(0, SLABWORDS)], sem1).wait()


def kernel(input_coords, num_atoms):
    b, t, mc = input_coords.shape
    n_max = mc // 3
    coords = input_coords.reshape(b * t, n_max * 3)
    na = num_atoms.reshape(b * t).astype(jnp.int32)
    na = jnp.pad(na, (0, 32 - b * t))

    mesh = plsc.VectorSubcoreMesh(core_axis_name="c", subcore_axis_name="s")
    run = pl.kernel(
        _body,
        out_type=jax.ShapeDtypeStruct((NT * SLABWORDS,), jnp.float32),
        mesh=mesh,
        compiler_params=pltpu.CompilerParams(needs_layout_passes=False),
        scratch_types=[
            pltpu.VMEM((3 * NATOM,), jnp.float32),
            pltpu.VMEM((32,), jnp.int32),
            pltpu.VMEM((NATOM + 32,), jnp.int32),
            pltpu.VMEM((SLABWORDS,), jnp.float32),
            pltpu.VMEM((SLABWORDS,), jnp.float32),
            pltpu.VMEM_SHARED((ZHALF,), jnp.float32),
            pltpu.SemaphoreType.DMA,
            pltpu.SemaphoreType.DMA,
            pltpu.SemaphoreType.DMA,
        ],
    )
    vols = run(coords, na)
    vols = vols.reshape(b, t, BOX, BOX, BOXP)[..., :BOX]
    return vols


# speedup vs baseline: 1.1823x; 1.0782x over previous
import jax
import jax.numpy as jnp
from jax import lax
from jax.experimental import pallas as pl
from jax.experimental.pallas import tpu as pltpu
from jax.experimental.pallas import tpu_sc as plsc

BOX = 120
D = 2
NMOL = 22
NATOM = 400
SLABW = 4
NSLAB = BOX // SLABW
BOXP = 128
PLANE = (BOX // 8) * 8 * BOXP
SLABWORDS = SLABW * PLANE
NT = NMOL * NSLAB
NC = 2
NS = 16
NW = NC * NS
KMAX = (NT + NW - 1) // NW
ZHALF = SLABWORDS // 2


def _body(coords_hbm, na_hbm, out_hbm, cv, na_v, sel, buf0, buf1, zsh,
          sem0, sem1, semz):
    c = lax.axis_index("c")
    s = lax.axis_index("s")
    wid = s * NC + c

    pltpu.sync_copy(na_hbm, na_v)

    lane = lax.iota(jnp.int32, 16)
    zero16 = jnp.zeros((16,), jnp.float32)
    oy1 = lane // 5
    oz1 = lane - oy1 * 5
    l2 = lane + 16
    oy2 = l2 // 5
    oz2 = l2 - oy2 * 5
    m_l2 = l2 < 25
    oy1f = oy1.astype(jnp.float32)
    oz1f = oz1.astype(jnp.float32)
    oy2f = oy2.astype(jnp.float32)
    oz2f = oz2.astype(jnp.float32)
    i0 = jnp.zeros((16,), jnp.int32)
    i1 = i0 + 1
    i2 = i0 + 2

    stripe = ZHALF // NS

    def z0(j, carry):
        buf0[pl.ds(j * 16, 16)] = zero16
        return carry

    lax.fori_loop(0, stripe // 16, z0, 0)
    pltpu.sync_copy(buf0.at[pl.ds(0, stripe)], zsh.at[pl.ds(s * stripe, stripe)])
    plsc.subcore_barrier()

    def process(k, t, buf, sem):
        if True:
            m = t // NSLAB
            slab = t - m * NSLAB
            x0 = slab * SLABW

            @pl.when((k == 0) | (slab == 0))
            def _():
                pltpu.sync_copy(coords_hbm.at[m], cv)

            @pl.when(k >= 2)
            def _():
                pltpu.make_async_copy(
                    buf, out_hbm.at[pl.ds(0, SLABWORDS)], sem).wait()
            hz = pltpu.async_copy(zsh, buf.at[pl.ds(0, ZHALF)], semz)
            na = plsc.load_gather(na_v, [i0 + m])

            def zb(j, carry):
                for u in range(8):
                    buf[pl.ds(ZHALF + (j * 8 + u) * 16, 16)] = zero16
                return carry

            lax.fori_loop(0, (SLABWORDS - ZHALF) // 128, zb, 0)

            def selb(i, cnt):
                ids = i * 16 + lane
                xs = plsc.load_gather(cv, [ids * 3])
                cx = xs.astype(jnp.int32)
                msk = (ids < na) & (cx >= x0 - D) & (cx <= x0 + SLABW - 1 + D)
                plsc.store_compressed(sel.at[pl.ds(cnt, 16)], ids, mask=msk)
                pc = plsc.all_reduce_population_count(msk)
                return cnt + jnp.max(pc)

            cnt = lax.fori_loop(0, NATOM // 16, selb, 0)
            hz.wait()

            def group(g, carry):
                gb = g * 16
                gid3 = jnp.clip(plsc.load_gather(sel, [gb + lane]),
                                0, NATOM - 1) * 3
                gx = plsc.load_gather(cv, [gid3])
                gy = plsc.load_gather(cv, [gid3 + 1])
                gz = plsc.load_gather(cv, [gid3 + 2])
                gcx = jnp.clip(gx.astype(jnp.int32), D, BOX - 1 - D)
                gcy = jnp.clip(gy.astype(jnp.int32), D, BOX - 1 - D)
                gcz = jnp.clip(gz.astype(jnp.int32), D, BOX - 1 - D)
                gdx0 = gx - gcx.astype(jnp.float32) + jnp.float32(D)
                gdy0 = gy - gcy.astype(jnp.float32) + jnp.float32(D)
                gdz0 = gz - gcz.astype(jnp.float32) + jnp.float32(D)
                gwx = []
                for ox in range(2 * D + 1):
                    dx = gdx0 - jnp.float32(ox)
                    gwx.append(jnp.exp(-(dx * dx)))
                gxrel0 = gcx - D - x0

                def take(v, jv):
                    return v[jv]

                def atom(j, carry):
                    jv = i0 + j
                    cyS = take(gcy, jv)
                    czS = take(gcz, jv)
                    dy0S = take(gdy0, jv)
                    dz0S = take(gdz0, jv)
                    xrel0S = take(gxrel0, jv)
                    dy1 = dy0S - oy1f
                    dz1 = dz0S - oz1f
                    wyz1 = jnp.exp(-(dy1 * dy1 + dz1 * dz1))
                    dy2 = dy0S - oy2f
                    dz2 = dz0S - oz2f
                    wyz2 = jnp.exp(-(dy2 * dy2 + dz2 * dz2))
                    ny1 = cyS - D + oy1
                    nz1 = czS - D + oz1
                    ny2 = cyS - D + oy2
                    nz2 = czS - D + oz2
                    base1 = (ny1 >> 3) * 1024 + (ny1 & 7) * BOXP + nz1
                    base2 = (ny2 >> 3) * 1024 + (ny2 & 7) * BOXP + nz2
                    for ox in range(2 * D + 1):
                        wx = take(gwx[ox], jv)
                        xr = xrel0S + ox
                        ms = xr.astype(jnp.uint32) < SLABW
                        row = jnp.minimum(xr.astype(jnp.uint32),
                                          SLABW - 1).astype(jnp.int32) * PLANE
                        plsc.addupdate_scatter(buf, [row + base1], wx * wyz1,
                                               mask=ms)
                        plsc.addupdate_scatter(buf, [row + base2], wx * wyz2,
                                               mask=ms & m_l2)
                    return carry

                lax.fori_loop(0, jnp.minimum(cnt - gb, 16), atom, 0)
                return carry

            lax.fori_loop(0, (cnt + 15) // 16, group, 0)

            pltpu.async_copy(
                buf, out_hbm.at[pl.ds(t * SLABWORDS, SLABWORDS)], sem)

    start = 20 * wid + jnp.minimum(wid, 20)
    ntask = jnp.where(wid < 20, 21, 20)

    def kstep(k, carry):
        t = start + k
        par = lax.rem(k, 2)

        @pl.when((k < ntask) & (par == 0))
        def _():
            process(k, t, buf0, sem0)

        @pl.when((k < ntask) & (par == 1))
        def _():
            process(k, t, buf1, sem1)

        return carry

    lax.fori_loop(0, KMAX, kstep, 0)
    pltpu.make_async_copy(buf0, out_hbm.at[pl.ds(0, SLABWORDS)], sem0).wait()
    pltpu.make_async_copy(buf1, out_hbm.at[pl.ds(0, SLABWORDS)], sem1).wait()


def kernel(input_coords, num_atoms):
    b, t, mc = input_coords.shape
    n_max = mc // 3
    coords = input_coords.reshape(b * t, n_max * 3)
    na = num_atoms.reshape(b * t).astype(jnp.int32)
    na = jnp.pad(na, (0, 32 - b * t))

    mesh = plsc.VectorSubcoreMesh(core_axis_name="c", subcore_axis_name="s")
    run = pl.kernel(
        _body,
        out_type=jax.ShapeDtypeStruct((NT * SLABWORDS,), jnp.float32),
        mesh=mesh,
        compiler_params=pltpu.CompilerParams(needs_layout_passes=False),
        scratch_types=[
            pltpu.VMEM((3 * NATOM,), jnp.float32),
            pltpu.VMEM((32,), jnp.int32),
            pltpu.VMEM((NATOM + 32,), jnp.int32),
            pltpu.VMEM((SLABWORDS,), jnp.float32),
            pltpu.VMEM((SLABWORDS,), jnp.float32),
            pltpu.VMEM_SHARED((ZHALF,), jnp.float32),
            pltpu.SemaphoreType.DMA,
            pltpu.SemaphoreType.DMA,
            pltpu.SemaphoreType.DMA,
        ],
    )
    vols = run(coords, na)
    vols = vols.reshape(b, t, BOX, BOX, BOXP)[..., :BOX]
    return vols


# speedup vs baseline: 1.1887x; 1.0054x over previous
import jax
import jax.numpy as jnp
from jax import lax
from jax.experimental import pallas as pl
from jax.experimental.pallas import tpu as pltpu
from jax.experimental.pallas import tpu_sc as plsc

BOX = 120
D = 2
NMOL = 22
NATOM = 400
SLABW = 4
NSLAB = BOX // SLABW
BOXP = 128
PLANE = (BOX // 8) * 8 * BOXP
SLABWORDS = SLABW * PLANE
NT = NMOL * NSLAB
NC = 2
NS = 16
NW = NC * NS
KMAX = (NT + NW - 1) // NW
ZHALF = SLABWORDS // 2


def _body(coords_hbm, na_hbm, out_hbm, cv, na_v, sel, buf0, buf1, zsh,
          sem0, sem1, semz):
    c = lax.axis_index("c")
    s = lax.axis_index("s")
    wid = s * NC + c

    pltpu.sync_copy(na_hbm, na_v)

    lane = lax.iota(jnp.int32, 16)
    zero16 = jnp.zeros((16,), jnp.float32)
    oy1 = lane // 5
    oz1 = lane - oy1 * 5
    l2 = lane + 16
    oy2 = l2 // 5
    oz2 = l2 - oy2 * 5
    m_l2 = l2 < 25
    oy1f = oy1.astype(jnp.float32)
    oz1f = oz1.astype(jnp.float32)
    oy2f = oy2.astype(jnp.float32)
    oz2f = oz2.astype(jnp.float32)
    i0 = jnp.zeros((16,), jnp.int32)
    i1 = i0 + 1
    i2 = i0 + 2

    stripe = ZHALF // NS

    def z0(j, carry):
        buf0[pl.ds(j * 16, 16)] = zero16
        return carry

    lax.fori_loop(0, stripe // 16, z0, 0)
    pltpu.sync_copy(buf0.at[pl.ds(0, stripe)], zsh.at[pl.ds(s * stripe, stripe)])
    plsc.subcore_barrier()

    def process(k, t, buf, sem):
        if True:
            m = t // NSLAB
            slab = t - m * NSLAB
            x0 = slab * SLABW

            @pl.when((k == 0) | (slab == 0))
            def _():
                pltpu.sync_copy(coords_hbm.at[m], cv)

            @pl.when(k >= 2)
            def _():
                pltpu.make_async_copy(
                    buf, out_hbm.at[pl.ds(0, SLABWORDS)], sem).wait()
            hz = pltpu.async_copy(zsh, buf.at[pl.ds(0, ZHALF)], semz)
            na = plsc.load_gather(na_v, [i0 + m])

            def zb(j, carry):
                for u in range(8):
                    buf[pl.ds(ZHALF + (j * 8 + u) * 16, 16)] = zero16
                return carry

            lax.fori_loop(0, (SLABWORDS - ZHALF) // 128, zb, 0)

            def selb(i, cnt):
                ids = i * 16 + lane
                xs = plsc.load_gather(cv, [ids * 3])
                cx = xs.astype(jnp.int32)
                msk = (ids < na) & (cx >= x0 - D) & (cx <= x0 + SLABW - 1 + D)
                plsc.store_compressed(sel.at[pl.ds(cnt, 16)], ids, mask=msk)
                pc = plsc.all_reduce_population_count(msk)
                return cnt + pc[0]

            nsweep = (na[0] + 15) // 16
            cnt = lax.fori_loop(0, nsweep, selb, 0)
            hz.wait()

            def group(g, carry):
                gb = g * 16
                gid3 = jnp.clip(plsc.load_gather(sel, [gb + lane]),
                                0, NATOM - 1) * 3
                gx = plsc.load_gather(cv, [gid3])
                gy = plsc.load_gather(cv, [gid3 + 1])
                gz = plsc.load_gather(cv, [gid3 + 2])
                gcx = jnp.clip(gx.astype(jnp.int32), D, BOX - 1 - D)
                gcy = jnp.clip(gy.astype(jnp.int32), D, BOX - 1 - D)
                gcz = jnp.clip(gz.astype(jnp.int32), D, BOX - 1 - D)
                gdx0 = gx - gcx.astype(jnp.float32) + jnp.float32(D)
                gdy0 = gy - gcy.astype(jnp.float32) + jnp.float32(D)
                gdz0 = gz - gcz.astype(jnp.float32) + jnp.float32(D)
                gwx = []
                for ox in range(2 * D + 1):
                    dx = gdx0 - jnp.float32(ox)
                    gwx.append(jnp.exp(-(dx * dx)))
                gxrel0 = gcx - D - x0

                def take(v, jv):
                    return v[jv]

                def atom(j, carry):
                    jv = i0 + j
                    cyS = take(gcy, jv)
                    czS = take(gcz, jv)
                    dy0S = take(gdy0, jv)
                    dz0S = take(gdz0, jv)
                    xrel0S = take(gxrel0, jv)
                    dy1 = dy0S - oy1f
                    dz1 = dz0S - oz1f
                    wyz1 = jnp.exp(-(dy1 * dy1 + dz1 * dz1))
                    dy2 = dy0S - oy2f
                    dz2 = dz0S - oz2f
                    wyz2 = jnp.exp(-(dy2 * dy2 + dz2 * dz2))
                    ny1 = cyS - D + oy1
                    nz1 = czS - D + oz1
                    ny2 = cyS - D + oy2
                    nz2 = czS - D + oz2
                    base1 = (ny1 >> 3) * 1024 + (ny1 & 7) * BOXP + nz1
                    base2 = (ny2 >> 3) * 1024 + (ny2 & 7) * BOXP + nz2
                    for ox in range(2 * D + 1):
                        wx = take(gwx[ox], jv)
                        xr = xrel0S + ox
                        ms = xr.astype(jnp.uint32) < SLABW
                        row = jnp.minimum(xr.astype(jnp.uint32),
                                          SLABW - 1).astype(jnp.int32) * PLANE
                        plsc.addupdate_scatter(buf, [row + base1], wx * wyz1,
                                               mask=ms)
                        plsc.addupdate_scatter(buf, [row + base2], wx * wyz2,
                                               mask=ms & m_l2)
                    return carry

                lax.fori_loop(0, jnp.minimum(cnt - gb, 16), atom, 0)
                return carry

            lax.fori_loop(0, (cnt + 15) // 16, group, 0)

            pltpu.async_copy(
                buf, out_hbm.at[pl.ds(t * SLABWORDS, SLABWORDS)], sem)

    start = 20 * wid + jnp.minimum(wid, 20)
    ntask = jnp.where(wid < 20, 21, 20)

    def kstep(k, carry):
        t = start + k
        par = lax.rem(k, 2)

        @pl.when((k < ntask) & (par == 0))
        def _():
            process(k, t, buf0, sem0)

        @pl.when((k < ntask) & (par == 1))
        def _():
            process(k, t, buf1, sem1)

        return carry

    lax.fori_loop(0, KMAX, kstep, 0)
    pltpu.make_async_copy(buf0, out_hbm.at[pl.ds(0, SLABWORDS)], sem0).wait()
    pltpu.make_async_copy(buf1, out_hbm.at[pl.ds(0, SLABWORDS)], sem1).wait()


def kernel(input_coords, num_atoms):
    b, t, mc = input_coords.shape
    n_max = mc // 3
    coords = input_coords.reshape(b * t, n_max * 3)
    na = num_atoms.reshape(b * t).astype(jnp.int32)
    na = jnp.pad(na, (0, 32 - b * t))

    mesh = plsc.VectorSubcoreMesh(core_axis_name="c", subcore_axis_name="s")
    run = pl.kernel(
        _body,
        out_type=jax.ShapeDtypeStruct((NT * SLABWORDS,), jnp.float32),
        mesh=mesh,
        compiler_params=pltpu.CompilerParams(needs_layout_passes=False),
        scratch_types=[
            pltpu.VMEM((3 * NATOM,), jnp.float32),
            pltpu.VMEM((32,), jnp.int32),
            pltpu.VMEM((NATOM + 32,), jnp.int32),
            pltpu.VMEM((SLABWORDS,), jnp.float32),
            pltpu.VMEM((SLABWORDS,), jnp.float32),
            pltpu.VMEM_SHARED((ZHALF,), jnp.float32),
            pltpu.SemaphoreType.DMA,
            pltpu.SemaphoreType.DMA,
            pltpu.SemaphoreType.DMA,
        ],
    )
    vols = run(coords, na)
    vols = vols.reshape(b, t, BOX, BOX, BOXP)[..., :BOX]
    return vols


# speedup vs baseline: 1.2087x; 1.0168x over previous
import jax
import jax.numpy as jnp
from jax import lax
from jax.experimental import pallas as pl
from jax.experimental.pallas import tpu as pltpu
from jax.experimental.pallas import tpu_sc as plsc

BOX = 120
D = 2
NMOL = 22
NATOM = 400
SLABW = 4
NSLAB = BOX // SLABW
BOXP = 128
PLANE = (BOX // 8) * 8 * BOXP
SLABWORDS = SLABW * PLANE
NT = NMOL * NSLAB
NC = 2
NS = 16
NW = NC * NS
KMAX = (NT + NW - 1) // NW
ZHALF = SLABWORDS // 2


def _body(cin_hbm, out_hbm, cv, na_v, sel, buf0, buf1, zsh,
          sem0, sem1, semz):
    c = lax.axis_index("c")
    s = lax.axis_index("s")
    wid = s * NC + c

    pltpu.sync_copy(cin_hbm.at[pl.ds(NMOL * 3 * NATOM, 32)], na_v)

    lane = lax.iota(jnp.int32, 16)
    zero16 = jnp.zeros((16,), jnp.float32)
    oy1 = lane // 5
    oz1 = lane - oy1 * 5
    l2 = lane + 16
    oy2 = l2 // 5
    oz2 = l2 - oy2 * 5
    m_l2 = l2 < 25
    oy1f = oy1.astype(jnp.float32)
    oz1f = oz1.astype(jnp.float32)
    oy2f = oy2.astype(jnp.float32)
    oz2f = oz2.astype(jnp.float32)
    i0 = jnp.zeros((16,), jnp.int32)
    i1 = i0 + 1
    i2 = i0 + 2

    stripe = ZHALF // NS

    def z0(j, carry):
        buf0[pl.ds(j * 16, 16)] = zero16
        return carry

    lax.fori_loop(0, stripe // 16, z0, 0)
    pltpu.sync_copy(buf0.at[pl.ds(0, stripe)], zsh.at[pl.ds(s * stripe, stripe)])
    plsc.subcore_barrier()

    def process(k, t, buf, sem):
        if True:
            m = t // NSLAB
            slab = t - m * NSLAB
            x0 = slab * SLABW

            @pl.when((k == 0) | (slab == 0))
            def _():
                pltpu.sync_copy(cin_hbm.at[pl.ds(m * 3 * NATOM, 3 * NATOM)],
                                cv)

            @pl.when(k >= 2)
            def _():
                pltpu.make_async_copy(
                    buf, out_hbm.at[pl.ds(0, SLABWORDS)], sem).wait()
            hz = pltpu.async_copy(zsh, buf.at[pl.ds(0, ZHALF)], semz)
            na = plsc.bitcast(plsc.load_gather(na_v, [i0 + m]), jnp.int32)

            def zb(j, carry):
                for u in range(8):
                    buf[pl.ds(ZHALF + (j * 8 + u) * 16, 16)] = zero16
                return carry

            lax.fori_loop(0, (SLABWORDS - ZHALF) // 128, zb, 0)

            def selb(i, cnt):
                ids = i * 16 + lane
                xs = plsc.load_gather(cv, [ids * 3])
                cx = xs.astype(jnp.int32)
                msk = (ids < na) & (cx >= x0 - D) & (cx <= x0 + SLABW - 1 + D)
                plsc.store_compressed(sel.at[pl.ds(cnt, 16)], ids, mask=msk)
                pc = plsc.all_reduce_population_count(msk)
                return cnt + pc[0]

            nsweep = (na[0] + 15) // 16
            cnt = lax.fori_loop(0, nsweep, selb, 0)
            hz.wait()

            def group(g, carry):
                gb = g * 16
                gid3 = jnp.clip(plsc.load_gather(sel, [gb + lane]),
                                0, NATOM - 1) * 3
                gx = plsc.load_gather(cv, [gid3])
                gy = plsc.load_gather(cv, [gid3 + 1])
                gz = plsc.load_gather(cv, [gid3 + 2])
                gcx = jnp.clip(gx.astype(jnp.int32), D, BOX - 1 - D)
                gcy = jnp.clip(gy.astype(jnp.int32), D, BOX - 1 - D)
                gcz = jnp.clip(gz.astype(jnp.int32), D, BOX - 1 - D)
                gdx0 = gx - gcx.astype(jnp.float32) + jnp.float32(D)
                gdy0 = gy - gcy.astype(jnp.float32) + jnp.float32(D)
                gdz0 = gz - gcz.astype(jnp.float32) + jnp.float32(D)
                gwx = []
                for ox in range(2 * D + 1):
                    dx = gdx0 - jnp.float32(ox)
                    gwx.append(jnp.exp(-(dx * dx)))
                gxrel0 = gcx - D - x0

                def take(v, jv):
                    return v[jv]

                def atom(j, carry):
                    jv = i0 + j
                    cyS = take(gcy, jv)
                    czS = take(gcz, jv)
                    dy0S = take(gdy0, jv)
                    dz0S = take(gdz0, jv)
                    xrel0S = take(gxrel0, jv)
                    dy1 = dy0S - oy1f
                    dz1 = dz0S - oz1f
                    wyz1 = jnp.exp(-(dy1 * dy1 + dz1 * dz1))
                    dy2 = dy0S - oy2f
                    dz2 = dz0S - oz2f
                    wyz2 = jnp.exp(-(dy2 * dy2 + dz2 * dz2))
                    ny1 = cyS - D + oy1
                    nz1 = czS - D + oz1
                    ny2 = cyS - D + oy2
                    nz2 = czS - D + oz2
                    base1 = (ny1 >> 3) * 1024 + (ny1 & 7) * BOXP + nz1
                    base2 = (ny2 >> 3) * 1024 + (ny2 & 7) * BOXP + nz2
                    for ox in range(2 * D + 1):
                        wx = take(gwx[ox], jv)
                        xr = xrel0S + ox
                        ms = xr.astype(jnp.uint32) < SLABW
                        row = jnp.minimum(xr.astype(jnp.uint32),
                                          SLABW - 1).astype(jnp.int32) * PLANE
                        plsc.addupdate_scatter(buf, [row + base1], wx * wyz1,
                                               mask=ms)
                        plsc.addupdate_scatter(buf, [row + base2], wx * wyz2,
                                               mask=ms & m_l2)
                    return carry

                lax.fori_loop(0, jnp.minimum(cnt - gb, 16), atom, 0)
                return carry

            lax.fori_loop(0, (cnt + 15) // 16, group, 0)

            pltpu.async_copy(
                buf, out_hbm.at[pl.ds(t * SLABWORDS, SLABWORDS)], sem)

    start = 20 * wid + jnp.minimum(wid, 20)
    ntask = jnp.where(wid < 20, 21, 20)

    def kstep(k, carry):
        t = start + k
        par = lax.rem(k, 2)

        @pl.when((k < ntask) & (par == 0))
        def _():
            process(k, t, buf0, sem0)

        @pl.when((k < ntask) & (par == 1))
        def _():
            process(k, t, buf1, sem1)

        return carry

    lax.fori_loop(0, KMAX, kstep, 0)
    pltpu.make_async_copy(buf0, out_hbm.at[pl.ds(0, SLABWORDS)], sem0).wait()
    pltpu.make_async_copy(buf1, out_hbm.at[pl.ds(0, SLABWORDS)], sem1).wait()


def kernel(input_coords, num_atoms):
    b, t, mc = input_coords.shape
    n_max = mc // 3
    na = num_atoms.reshape(b * t).astype(jnp.int32)
    na = jnp.pad(na, (0, 32 - b * t))
    cin = jnp.concatenate([input_coords.reshape(b * t * mc),
                           jax.lax.bitcast_convert_type(na, jnp.float32)])

    mesh = plsc.VectorSubcoreMesh(core_axis_name="c", subcore_axis_name="s")
    run = pl.kernel(
        _body,
        out_type=jax.ShapeDtypeStruct((NT * SLABWORDS,), jnp.float32),
        mesh=mesh,
        compiler_params=pltpu.CompilerParams(needs_layout_passes=False),
        scratch_types=[
            pltpu.VMEM((3 * NATOM,), jnp.float32),
            pltpu.VMEM((32,), jnp.float32),
            pltpu.VMEM((NATOM + 32,), jnp.int32),
            pltpu.VMEM((SLABWORDS,), jnp.float32),
            pltpu.VMEM((SLABWORDS,), jnp.float32),
            pltpu.VMEM_SHARED((ZHALF,), jnp.float32),
            pltpu.SemaphoreType.DMA,
            pltpu.SemaphoreType.DMA,
            pltpu.SemaphoreType.DMA,
        ],
    )
    vols = run(cin)
    vols = vols.reshape(b, t, BOX, BOX, BOXP)[..., :BOX]
    return vols
